# baseline (speedup 1.0000x reference)
import functools

import jax
import jax.numpy as jnp
from jax import lax
from jax.experimental import pallas as pl
from jax.experimental.pallas import tpu as pltpu
from jax.experimental.pallas import tpu_sc as plsc

VOCAB = 100000
D_MODEL = 64
BATCH = 4096
SEQ = 200

N = BATCH * SEQ
NUM_WORKERS = 32
PER_WORKER = N // NUM_WORKERS
CHUNK = 800
NUM_CHUNKS = PER_WORKER // CHUNK

_mesh = plsc.VectorSubcoreMesh(core_axis_name="c", subcore_axis_name="s")


BATCH_PER_CHUNK = CHUNK // SEQ


@functools.partial(
    pl.kernel,
    mesh=_mesh,
    out_type=jax.ShapeDtypeStruct((BATCH, SEQ, D_MODEL), jnp.float32),
    scratch_types=[
        pltpu.VMEM((CHUNK,), jnp.int32),
        pltpu.VMEM((CHUNK,), jnp.int32),
        pltpu.VMEM((CHUNK, D_MODEL), jnp.float32),
        pltpu.VMEM((CHUNK, D_MODEL), jnp.float32),
        pltpu.SemaphoreType.DMA,
        pltpu.SemaphoreType.DMA,
    ],
    compiler_params=pltpu.CompilerParams(use_tc_tiling_on_sc=False),
)
def _embedding_lookup(idx_hbm, table_hbm, out_hbm,
                      idx0, idx1, rows0, rows1, sem0, sem1):
    wid = lax.axis_index("s") * 2 + lax.axis_index("c")
    base = wid * PER_WORKER

    def _write_out(rows, flat_start):
        b0 = flat_start // SEQ
        for k in range(BATCH_PER_CHUNK):
            pltpu.sync_copy(rows.at[pl.ds(k * SEQ, SEQ)], out_hbm.at[b0 + k])

    pltpu.sync_copy(idx_hbm.at[pl.ds(base, CHUNK)], idx0)
    pltpu.async_copy(table_hbm.at[idx0], rows0, sem0)

    def body(k, carry):
        g0 = 2 * k
        s1 = base + (g0 + 1) * CHUNK
        pltpu.sync_copy(idx_hbm.at[pl.ds(s1, CHUNK)], idx1)
        pltpu.async_copy(table_hbm.at[idx1], rows1, sem1)
        pltpu.make_async_copy(table_hbm.at[idx0], rows0, sem0).wait()
        _write_out(rows0, base + g0 * CHUNK)
        s2 = base + lax.rem(g0 + 2, NUM_CHUNKS) * CHUNK
        pltpu.sync_copy(idx_hbm.at[pl.ds(s2, CHUNK)], idx0)
        pltpu.async_copy(table_hbm.at[idx0], rows0, sem0)
        pltpu.make_async_copy(table_hbm.at[idx1], rows1, sem1).wait()
        _write_out(rows1, s1)
        return carry

    lax.fori_loop(0, NUM_CHUNKS // 2, body, 0)
    pltpu.make_async_copy(table_hbm.at[idx0], rows0, sem0).wait()


def kernel(indices, weight):
    flat_idx = indices.reshape(N)
    return _embedding_lookup(flat_idx, weight)


# speedup vs baseline: 1.3238x; 1.3238x over previous
import functools

import jax
import jax.numpy as jnp
from jax import lax
from jax.experimental import pallas as pl
from jax.experimental.pallas import tpu as pltpu
from jax.experimental.pallas import tpu_sc as plsc

VOCAB = 100000
D_MODEL = 64
D_PAD = 128
BATCH = 4096
SEQ = 200

N = BATCH * SEQ
NUM_WORKERS = 32
PER_WORKER = N // NUM_WORKERS
CHUNK = 400
NUM_CHUNKS = PER_WORKER // CHUNK
BATCH_PER_CHUNK = CHUNK // SEQ

_mesh = plsc.VectorSubcoreMesh(core_axis_name="c", subcore_axis_name="s")


@functools.partial(
    pl.kernel,
    mesh=_mesh,
    out_type=jax.ShapeDtypeStruct((BATCH, SEQ, D_PAD), jnp.float32),
    scratch_types=[
        pltpu.VMEM((CHUNK,), jnp.int32),
        pltpu.VMEM((CHUNK,), jnp.int32),
        pltpu.VMEM((CHUNK, D_PAD), jnp.float32),
        pltpu.VMEM((CHUNK, D_PAD), jnp.float32),
        pltpu.SemaphoreType.DMA,
        pltpu.SemaphoreType.DMA,
    ],
    compiler_params=pltpu.CompilerParams(use_tc_tiling_on_sc=True),
)
def _embedding_lookup(idx_hbm, table_hbm, out_hbm,
                      idx0, idx1, rows0, rows1, sem0, sem1):
    wid = lax.axis_index("s") * 2 + lax.axis_index("c")
    base = wid * PER_WORKER

    def _write_out(rows, flat_start):
        b0 = flat_start // SEQ
        for k in range(BATCH_PER_CHUNK):
            pltpu.sync_copy(rows.at[pl.ds(k * SEQ, SEQ)], out_hbm.at[b0 + k])

    pltpu.sync_copy(idx_hbm.at[pl.ds(base, CHUNK)], idx0)
    pltpu.async_copy(table_hbm.at[idx0], rows0, sem0)

    def body(k, carry):
        g0 = 2 * k
        s1 = base + (g0 + 1) * CHUNK
        pltpu.sync_copy(idx_hbm.at[pl.ds(s1, CHUNK)], idx1)
        pltpu.async_copy(table_hbm.at[idx1], rows1, sem1)
        pltpu.make_async_copy(table_hbm.at[idx0], rows0, sem0).wait()
        _write_out(rows0, base + g0 * CHUNK)
        s2 = base + lax.rem(g0 + 2, NUM_CHUNKS) * CHUNK
        pltpu.sync_copy(idx_hbm.at[pl.ds(s2, CHUNK)], idx0)
        pltpu.async_copy(table_hbm.at[idx0], rows0, sem0)
        pltpu.make_async_copy(table_hbm.at[idx1], rows1, sem1).wait()
        _write_out(rows1, s1)
        return carry

    lax.fori_loop(0, NUM_CHUNKS // 2, body, 0)
    pltpu.make_async_copy(table_hbm.at[idx0], rows0, sem0).wait()


def kernel(indices, weight):
    flat_idx = indices.reshape(N)
    table = jnp.pad(weight, ((0, 0), (0, D_PAD - D_MODEL)))
    out = _embedding_lookup(flat_idx, table)
    return out[:, :, :D_MODEL]


# speedup vs baseline: 1.7172x; 1.2972x over previous
import functools

import jax
import jax.numpy as jnp
from jax import lax
from jax.experimental import pallas as pl
from jax.experimental.pallas import tpu as pltpu
from jax.experimental.pallas import tpu_sc as plsc

VOCAB = 100000
D_MODEL = 64
D_PAD = 128
BATCH = 4096
SEQ = 200

N = BATCH * SEQ
NUM_WORKERS = 32
PER_WORKER = N // NUM_WORKERS
CHUNK = 400
NUM_CHUNKS = PER_WORKER // CHUNK
BATCH_PER_CHUNK = CHUNK // SEQ

_mesh = plsc.VectorSubcoreMesh(core_axis_name="c", subcore_axis_name="s")


@functools.partial(
    pl.kernel,
    mesh=_mesh,
    out_type=jax.ShapeDtypeStruct((BATCH, SEQ, D_PAD), jnp.float32),
    scratch_types=[
        pltpu.VMEM((CHUNK,), jnp.int32),
        pltpu.VMEM((CHUNK,), jnp.int32),
        pltpu.VMEM((CHUNK, D_MODEL), jnp.float32),
        pltpu.VMEM((CHUNK, D_MODEL), jnp.float32),
        pltpu.SemaphoreType.DMA,
        pltpu.SemaphoreType.DMA,
    ],
    compiler_params=pltpu.CompilerParams(use_tc_tiling_on_sc=False),
)
def _embedding_lookup(idx_hbm, table_hbm, out_hbm,
                      idx0, idx1, rows0, rows1, sem0, sem1):
    wid = lax.axis_index("s") * 2 + lax.axis_index("c")
    base = wid * PER_WORKER

    def _write_out(rows, flat_start):
        b0 = flat_start // SEQ
        for k in range(BATCH_PER_CHUNK):
            pltpu.sync_copy(rows.at[pl.ds(k * SEQ, SEQ)],
                            out_hbm.at[b0 + k, :, pl.ds(0, D_MODEL)])

    pltpu.sync_copy(idx_hbm.at[pl.ds(base, CHUNK)], idx0)
    pltpu.async_copy(table_hbm.at[idx0], rows0, sem0)

    def body(k, carry):
        g0 = 2 * k
        s1 = base + (g0 + 1) * CHUNK
        pltpu.sync_copy(idx_hbm.at[pl.ds(s1, CHUNK)], idx1)
        pltpu.async_copy(table_hbm.at[idx1], rows1, sem1)
        pltpu.make_async_copy(table_hbm.at[idx0], rows0, sem0).wait()
        _write_out(rows0, base + g0 * CHUNK)
        s2 = base + lax.rem(g0 + 2, NUM_CHUNKS) * CHUNK
        pltpu.sync_copy(idx_hbm.at[pl.ds(s2, CHUNK)], idx0)
        pltpu.async_copy(table_hbm.at[idx0], rows0, sem0)
        pltpu.make_async_copy(table_hbm.at[idx1], rows1, sem1).wait()
        _write_out(rows1, s1)
        return carry

    lax.fori_loop(0, NUM_CHUNKS // 2, body, 0)
    pltpu.make_async_copy(table_hbm.at[idx0], rows0, sem0).wait()


def kernel(indices, weight):
    flat_idx = indices.reshape(N)
    out = _embedding_lookup(flat_idx, weight)
    return out[:, :, :D_MODEL]


# speedup vs baseline: 1.7855x; 1.0398x over previous
import functools

import jax
import jax.numpy as jnp
from jax import lax
from jax.experimental import pallas as pl
from jax.experimental.pallas import tpu as pltpu
from jax.experimental.pallas import tpu_sc as plsc

VOCAB = 100000
D_MODEL = 64
D_PAD = 128
BATCH = 4096
SEQ = 200

N = BATCH * SEQ
NUM_WORKERS = 32
PER_WORKER = N // NUM_WORKERS
CHUNK = 800
NUM_CHUNKS = PER_WORKER // CHUNK
BATCH_PER_CHUNK = CHUNK // SEQ

_mesh = plsc.VectorSubcoreMesh(core_axis_name="c", subcore_axis_name="s")


@functools.partial(
    pl.kernel,
    mesh=_mesh,
    out_type=jax.ShapeDtypeStruct((BATCH, SEQ, D_PAD), jnp.float32),
    scratch_types=[
        pltpu.VMEM((PER_WORKER,), jnp.int32),
        pltpu.VMEM((CHUNK, D_MODEL), jnp.float32),
        pltpu.VMEM((CHUNK, D_MODEL), jnp.float32),
        pltpu.SemaphoreType.DMA,
        pltpu.SemaphoreType.DMA,
    ],
    compiler_params=pltpu.CompilerParams(use_tc_tiling_on_sc=False),
)
def _embedding_lookup(idx_hbm, table_hbm, out_hbm,
                      idx_v, rows0, rows1, sem0, sem1):
    wid = lax.axis_index("s") * 2 + lax.axis_index("c")
    base = wid * PER_WORKER

    def _write_out(rows, flat_start):
        b0 = flat_start // SEQ
        for k in range(BATCH_PER_CHUNK):
            pltpu.sync_copy(rows.at[pl.ds(k * SEQ, SEQ)],
                            out_hbm.at[b0 + k, :, pl.ds(0, D_MODEL)])

    def _gather(g, rows, sem):
        return pltpu.async_copy(
            table_hbm.at[idx_v.at[pl.ds(g * CHUNK, CHUNK)]], rows, sem)

    pltpu.sync_copy(idx_hbm.at[pl.ds(base, PER_WORKER)], idx_v)
    _gather(0, rows0, sem0)

    def body(k, carry):
        g0 = 2 * k
        _gather(g0 + 1, rows1, sem1)
        pltpu.make_async_copy(table_hbm.at[idx_v.at[pl.ds(0, CHUNK)]], rows0, sem0).wait()
        _write_out(rows0, base + g0 * CHUNK)
        _gather(lax.rem(g0 + 2, NUM_CHUNKS), rows0, sem0)
        pltpu.make_async_copy(table_hbm.at[idx_v.at[pl.ds(0, CHUNK)]], rows1, sem1).wait()
        _write_out(rows1, base + (g0 + 1) * CHUNK)
        return carry

    lax.fori_loop(0, NUM_CHUNKS // 2, body, 0)
    pltpu.make_async_copy(table_hbm.at[idx_v.at[pl.ds(0, CHUNK)]], rows0, sem0).wait()


def kernel(indices, weight):
    flat_idx = indices.reshape(N)
    out = _embedding_lookup(flat_idx, weight)
    return out[:, :, :D_MODEL]
